# baseline (device time: 1319845 ns/iter reference)
import jax
import jax.numpy as jnp
from jax import lax
from jax.experimental import pallas as pl
from jax.experimental.pallas import tpu as pltpu

NDEV = 4
NSUB_N = 2
NHOP = NDEV - 1


def kernel(O, Wo):
    B, S, H, D = O.shape
    K = H * D
    N = Wo.shape[1]
    S_out = S // NDEV
    NQ = N // NSUB_N
    NQQ = NQ // 2
    NSUB = B * NSUB_N
    NMSG = NHOP * NSUB

    O3 = O.reshape(B, S, K).astype(jnp.bfloat16)
    W_b = Wo.astype(jnp.bfloat16)

    def body(o_ref, w_ref, out_ref, recv_ref,
             o_t, va, res, vo, ssems, rsems, csems):
        my = lax.axis_index("i")
        left = (my + NDEV - 1) % NDEV
        right = (my + 1) % NDEV

        barrier = pltpu.get_barrier_semaphore()
        for nbr in (left, right):
            pl.semaphore_signal(barrier, inc=1, device_id=(nbr,),
                                device_id_type=pl.DeviceIdType.MESH)
        pl.semaphore_wait(barrier, 2)

        def cp(src, dst, slot):
            c = pltpu.make_async_copy(src, dst, csems.at[slot])
            c.start()
            return c

        def desc(g, src):
            return pltpu.make_async_remote_copy(
                src_ref=src,
                dst_ref=recv_ref.at[g],
                send_sem=ssems.at[g],
                recv_sem=rsems.at[g],
                device_id=(right,),
                device_id_type=pl.DeviceIdType.MESH,
            )

        def send(g, p):
            desc(g, res.at[p]).start()

        def wait_send(g):
            desc(g, res.at[0]).wait_send()

        def wait_recv(g):
            desc(g, res.at[0]).wait_recv()

        def load_o(b, c):
            return cp(o_ref.at[b, pl.ds(c * S_out, S_out), :], o_t, 0)

        def partial(lo, width):
            return jnp.dot(o_t[...], w_ref[:, lo:lo + width],
                           preferred_element_type=jnp.float32)

        c0 = (my + NDEV - 1) % NDEV

        def hop0_b(b, carry):
            load_o(b, c0).wait()
            for nq in range(NSUB_N):
                g = b * NSUB_N + nq
                p = g % 2

                @pl.when(g >= 2)
                def _ws(g=g):
                    wait_send(g - 2)

                res[p, ...] = partial(nq * NQ, NQ).astype(jnp.bfloat16)
                send(g, p)
            return carry

        lax.fori_loop(0, B, hop0_b, 0)

        for h in (1, 2):
            c = (my + NDEV - 1 - h) % NDEV

            def hop_b(b, _, h=h, c=c):
                load_o(b, c).wait()
                for nq in range(NSUB_N):
                    g = h * NSUB + b * NSUB_N + nq
                    p = g % 2
                    wait_send(g - 2)
                    wait_recv(g - NSUB)
                    cp(recv_ref.at[g - NSUB], va, 1).wait()
                    res[p, ...] = (va[...].astype(jnp.float32)
                                   + partial(nq * NQ, NQ)).astype(jnp.bfloat16)
                    send(g, p)
                return _

            lax.fori_loop(0, B, hop_b, 0)

        def final_b(b, _):
            load_o(b, my).wait()
            for nq in range(NSUB_N):
                g = 2 * NSUB + b * NSUB_N + nq
                wait_recv(g)
                cp(recv_ref.at[g], va, 1).wait()
                for qq in range(2):
                    lo = nq * NQ + qq * NQQ
                    vo[...] = (va[:, qq * NQQ:(qq + 1) * NQQ].astype(jnp.float32)
                               + partial(lo, NQQ))
                    cp(vo, out_ref.at[b, :, pl.ds(lo, NQQ)], 1).wait()
            return _

        lax.fori_loop(0, B, final_b, 0)

        wait_send(NMSG - 2)
        wait_send(NMSG - 1)

    out, _ = pl.pallas_call(
        body,
        out_shape=[
            jax.ShapeDtypeStruct((B, S_out, N), jnp.float32),
            jax.ShapeDtypeStruct((NMSG, S_out, NQ), jnp.bfloat16),
        ],
        in_specs=[
            pl.BlockSpec(memory_space=pl.ANY),
            pl.BlockSpec(memory_space=pltpu.VMEM),
        ],
        out_specs=[
            pl.BlockSpec(memory_space=pl.ANY),
            pl.BlockSpec(memory_space=pl.ANY),
        ],
        scratch_shapes=[
            pltpu.VMEM((S_out, K), jnp.bfloat16),
            pltpu.VMEM((S_out, NQ), jnp.bfloat16),
            pltpu.VMEM((2, S_out, NQ), jnp.bfloat16),
            pltpu.VMEM((S_out, NQQ), jnp.float32),
            pltpu.SemaphoreType.DMA((NMSG,)),
            pltpu.SemaphoreType.DMA((NMSG,)),
            pltpu.SemaphoreType.DMA((2,)),
        ],
        compiler_params=pltpu.CompilerParams(
            collective_id=0,
            vmem_limit_bytes=60 * 1024 * 1024,
        ),
    )(O3, W_b)
    return out


# device time: 1282654 ns/iter; 1.0290x vs baseline; 1.0290x over previous
import jax
import jax.numpy as jnp
from jax import lax
from jax.experimental import pallas as pl
from jax.experimental.pallas import tpu as pltpu

NDEV = 4
NSUB_N = 2
NHOP = NDEV - 1


def kernel(O, Wo):
    B, S, H, D = O.shape
    K = H * D
    N = Wo.shape[1]
    S_out = S // NDEV
    NQ = N // NSUB_N
    NQQ = NQ // 2
    NSUB = B * NSUB_N
    NMSG = NHOP * NSUB

    O3 = O.reshape(B, S, K).astype(jnp.bfloat16)
    W_b = Wo.astype(jnp.bfloat16)

    def body(o_ref, w_ref, out_ref, recv_ref,
             o_t, va, res, vo, ssems, rsems, csems):
        my = lax.axis_index("i")
        left = (my + NDEV - 1) % NDEV
        right = (my + 1) % NDEV

        barrier = pltpu.get_barrier_semaphore()
        for nbr in (left, right):
            pl.semaphore_signal(barrier, inc=1, device_id=(nbr,),
                                device_id_type=pl.DeviceIdType.MESH)
        pl.semaphore_wait(barrier, 2)

        def cp(src, dst, slot):
            c = pltpu.make_async_copy(src, dst, csems.at[slot])
            c.start()
            return c

        def desc(g, src):
            return pltpu.make_async_remote_copy(
                src_ref=src,
                dst_ref=recv_ref.at[g],
                send_sem=ssems.at[g],
                recv_sem=rsems.at[g],
                device_id=(right,),
                device_id_type=pl.DeviceIdType.MESH,
            )

        def send(g, p):
            desc(g, res.at[p]).start()

        def wait_send(g):
            desc(g, res.at[0]).wait_send()

        def wait_recv(g):
            desc(g, res.at[0]).wait_recv()

        def o_desc(b, c, slot):
            return pltpu.make_async_copy(
                o_ref.at[b, pl.ds(c * S_out, S_out), :], o_t.at[slot],
                csems.at[0])

        def partial(sb, lo, width):
            return jnp.dot(o_t[sb], w_ref[:, lo:lo + width],
                           preferred_element_type=jnp.float32)

        c0 = (my + NDEV - 1) % NDEV
        o_desc(0, c0, 0).start()

        def hop0_b(b, carry):
            sb = b % 2
            o_desc(b, c0, sb).wait()

            @pl.when(b + 1 < B)
            def _pf(b=b):
                o_desc(b + 1, c0, (b + 1) % 2).start()

            for nq in range(NSUB_N):
                g = b * NSUB_N + nq
                p = g % 4

                @pl.when(g >= 4)
                def _ws(g=g):
                    wait_send(g - 4)

                res[p, ...] = partial(sb, nq * NQ, NQ).astype(jnp.bfloat16)
                send(g, p)
            return carry

        lax.fori_loop(0, B, hop0_b, 0)

        for h in (1, 2):
            c = (my + NDEV - 1 - h) % NDEV
            o_desc(0, c, 0).start()

            def hop_b(b, carry, h=h, c=c):
                sb = b % 2
                o_desc(b, c, sb).wait()

                @pl.when(b + 1 < B)
                def _pf(b=b, c=c):
                    o_desc(b + 1, c, (b + 1) % 2).start()

                for nq in range(NSUB_N):
                    g = h * NSUB + b * NSUB_N + nq
                    p = g % 4
                    wait_send(g - 4)
                    wait_recv(g - NSUB)
                    cp(recv_ref.at[g - NSUB], va, 1).wait()
                    res[p, ...] = (va[...].astype(jnp.float32)
                                   + partial(sb, nq * NQ, NQ)
                                   ).astype(jnp.bfloat16)
                    send(g, p)
                return carry

            lax.fori_loop(0, B, hop_b, 0)

        o_desc(0, my, 0).start()

        def final_b(b, carry):
            sb = b % 2
            o_desc(b, my, sb).wait()

            @pl.when(b + 1 < B)
            def _pf(b=b):
                o_desc(b + 1, my, (b + 1) % 2).start()

            for nq in range(NSUB_N):
                g = 2 * NSUB + b * NSUB_N + nq
                wait_recv(g)
                cp(recv_ref.at[g], va, 1).wait()
                for qq in range(2):
                    lo = nq * NQ + qq * NQQ
                    vo[...] = (va[:, qq * NQQ:(qq + 1) * NQQ].astype(jnp.float32)
                               + partial(sb, lo, NQQ))
                    cp(vo, out_ref.at[b, :, pl.ds(lo, NQQ)], 1).wait()
            return carry

        lax.fori_loop(0, B, final_b, 0)

        for g in range(NMSG - 4, NMSG):
            wait_send(g)

    out, _ = pl.pallas_call(
        body,
        out_shape=[
            jax.ShapeDtypeStruct((B, S_out, N), jnp.float32),
            jax.ShapeDtypeStruct((NMSG, S_out, NQ), jnp.bfloat16),
        ],
        in_specs=[
            pl.BlockSpec(memory_space=pl.ANY),
            pl.BlockSpec(memory_space=pltpu.VMEM),
        ],
        out_specs=[
            pl.BlockSpec(memory_space=pl.ANY),
            pl.BlockSpec(memory_space=pl.ANY),
        ],
        scratch_shapes=[
            pltpu.VMEM((2, S_out, K), jnp.bfloat16),
            pltpu.VMEM((S_out, NQ), jnp.bfloat16),
            pltpu.VMEM((4, S_out, NQ), jnp.bfloat16),
            pltpu.VMEM((S_out, NQQ), jnp.float32),
            pltpu.SemaphoreType.DMA((NMSG,)),
            pltpu.SemaphoreType.DMA((NMSG,)),
            pltpu.SemaphoreType.DMA((2,)),
        ],
        compiler_params=pltpu.CompilerParams(
            collective_id=0,
            vmem_limit_bytes=63 * 1024 * 1024,
        ),
    )(O3, W_b)
    return out


# device time: 1266053 ns/iter; 1.0425x vs baseline; 1.0131x over previous
import jax
import jax.numpy as jnp
from jax import lax
from jax.experimental import pallas as pl
from jax.experimental.pallas import tpu as pltpu

NDEV = 4
NSUB_N = 2
NHOP = NDEV - 1


def kernel(O, Wo):
    B, S, H, D = O.shape
    K = H * D
    N = Wo.shape[1]
    S_out = S // NDEV
    NQ = N // NSUB_N
    NQQ = NQ // 2
    NSUB = B * NSUB_N
    NMSG = NHOP * NSUB

    O3 = O.reshape(B, S, K)
    W_b = Wo.astype(jnp.bfloat16)

    def body(o_ref, w_ref, out_ref, recv_ref,
             o_t, va, res, vo, ssems, rsems, csems):
        my = lax.axis_index("i")
        left = (my + NDEV - 1) % NDEV
        right = (my + 1) % NDEV

        barrier = pltpu.get_barrier_semaphore()
        for nbr in (left, right):
            pl.semaphore_signal(barrier, inc=1, device_id=(nbr,),
                                device_id_type=pl.DeviceIdType.MESH)
        pl.semaphore_wait(barrier, 2)

        def cp(src, dst, slot):
            c = pltpu.make_async_copy(src, dst, csems.at[slot])
            c.start()
            return c

        def desc(g, src):
            return pltpu.make_async_remote_copy(
                src_ref=src,
                dst_ref=recv_ref.at[g],
                send_sem=ssems.at[g],
                recv_sem=rsems.at[g],
                device_id=(right,),
                device_id_type=pl.DeviceIdType.MESH,
            )

        def send(g, p):
            desc(g, res.at[p]).start()

        def wait_send(g):
            desc(g, res.at[0]).wait_send()

        def wait_recv(g):
            desc(g, res.at[0]).wait_recv()

        def o_desc(b, c, slot):
            return pltpu.make_async_copy(
                o_ref.at[b, pl.ds(c * S_out, S_out), :], o_t.at[slot],
                csems.at[0])

        def partial(sb, lo, width):
            return jnp.dot(o_t[sb].astype(jnp.bfloat16),
                           w_ref[:, lo:lo + width],
                           preferred_element_type=jnp.float32)

        c0 = (my + NDEV - 1) % NDEV
        o_desc(0, c0, 0).start()

        def hop0_b(b, carry):
            sb = b % 2
            o_desc(b, c0, sb).wait()

            @pl.when(b + 1 < B)
            def _pf(b=b):
                o_desc(b + 1, c0, (b + 1) % 2).start()

            for nq in range(NSUB_N):
                g = b * NSUB_N + nq
                p = g % 3

                @pl.when(g >= 3)
                def _ws(g=g):
                    wait_send(g - 3)

                res[p, ...] = partial(sb, nq * NQ, NQ).astype(jnp.bfloat16)
                send(g, p)
            return carry

        lax.fori_loop(0, B, hop0_b, 0)

        for h in (1, 2):
            c = (my + NDEV - 1 - h) % NDEV
            o_desc(0, c, 0).start()

            def hop_b(b, carry, h=h, c=c):
                sb = b % 2
                o_desc(b, c, sb).wait()

                @pl.when(b + 1 < B)
                def _pf(b=b, c=c):
                    o_desc(b + 1, c, (b + 1) % 2).start()

                for nq in range(NSUB_N):
                    g = h * NSUB + b * NSUB_N + nq
                    p = g % 3
                    wait_send(g - 3)
                    wait_recv(g - NSUB)
                    cp(recv_ref.at[g - NSUB], va, 1).wait()
                    res[p, ...] = (va[...].astype(jnp.float32)
                                   + partial(sb, nq * NQ, NQ)
                                   ).astype(jnp.bfloat16)
                    send(g, p)
                return carry

            lax.fori_loop(0, B, hop_b, 0)

        o_desc(0, my, 0).start()

        def final_b(b, carry):
            sb = b % 2
            o_desc(b, my, sb).wait()

            @pl.when(b + 1 < B)
            def _pf(b=b):
                o_desc(b + 1, my, (b + 1) % 2).start()

            for nq in range(NSUB_N):
                g = 2 * NSUB + b * NSUB_N + nq
                wait_recv(g)
                cp(recv_ref.at[g], va, 1).wait()
                for qq in range(2):
                    lo = nq * NQ + qq * NQQ
                    vo[...] = (va[:, qq * NQQ:(qq + 1) * NQQ].astype(jnp.float32)
                               + partial(sb, lo, NQQ))
                    cp(vo, out_ref.at[b, :, pl.ds(lo, NQQ)], 1).wait()
            return carry

        lax.fori_loop(0, B, final_b, 0)

        for g in range(NMSG - 3, NMSG):
            wait_send(g)

    out, _ = pl.pallas_call(
        body,
        out_shape=[
            jax.ShapeDtypeStruct((B, S_out, N), jnp.float32),
            jax.ShapeDtypeStruct((NMSG, S_out, NQ), jnp.bfloat16),
        ],
        in_specs=[
            pl.BlockSpec(memory_space=pl.ANY),
            pl.BlockSpec(memory_space=pltpu.VMEM),
        ],
        out_specs=[
            pl.BlockSpec(memory_space=pl.ANY),
            pl.BlockSpec(memory_space=pl.ANY),
        ],
        scratch_shapes=[
            pltpu.VMEM((2, S_out, K), jnp.float32),
            pltpu.VMEM((S_out, NQ), jnp.bfloat16),
            pltpu.VMEM((3, S_out, NQ), jnp.bfloat16),
            pltpu.VMEM((S_out, NQQ), jnp.float32),
            pltpu.SemaphoreType.DMA((NMSG,)),
            pltpu.SemaphoreType.DMA((NMSG,)),
            pltpu.SemaphoreType.DMA((2,)),
        ],
        compiler_params=pltpu.CompilerParams(
            collective_id=0,
            vmem_limit_bytes=63 * 1024 * 1024,
        ),
    )(O3, W_b)
    return out
